# baseline (speedup 1.0000x reference)
import functools

import jax
import jax.numpy as jnp
from jax.experimental import pallas as pl
from jax.experimental.pallas import tpu as pltpu
from jax.experimental.pallas import tpu_sc as plsc

B, N, D, C = 512, 16384, 256, 64
K = 5
EPS = 1e-05

BN = 4096
NT = N // BN
TOPK = K + 1
PAGES = 16
IMIN = -2**31


def _prep_body(feat_ref, pred_ref, fbank_ref, sbank_ref, trg_ref,
               q_ref, sm_ref, fnew_ref, snew_ref, fbf_ref):
    f = feat_ref[...]
    nrm = jnp.maximum(jnp.sqrt(jnp.sum(f * f, axis=1, keepdims=True)), 1e-12)
    q = f / nrm
    q_ref[...] = q
    p = pred_ref[...]
    p = p - jnp.max(p, axis=1, keepdims=True)
    e = jnp.exp(p)
    sm = e / jnp.sum(e, axis=1, keepdims=True)
    sm_ref[...] = sm
    fnew_ref[...] = fbank_ref[...]
    snew_ref[:, 0:C] = sbank_ref[...]
    snew_ref[:, C:2 * C] = jnp.zeros((N, C), jnp.float32)

    def body(b, _):
        idx = trg_ref[b]
        fnew_ref[pl.ds(idx, 1), :] = q_ref[pl.ds(b, 1), :]
        snew_ref[pl.ds(idx, 1), 0:C] = sm_ref[pl.ds(b, 1), :]
        return 0

    jax.lax.fori_loop(0, B, body, 0)
    fbf_ref[...] = fnew_ref[...].astype(jnp.bfloat16)


def _prep(features, predictions, fea_bank, score_bank, trg_idx):
    return pl.pallas_call(
        _prep_body,
        grid=(),
        in_specs=[
            pl.BlockSpec((B, D), lambda: (0, 0)),
            pl.BlockSpec((B, C), lambda: (0, 0)),
            pl.BlockSpec((N, D), lambda: (0, 0)),
            pl.BlockSpec((N, C), lambda: (0, 0)),
            pl.BlockSpec(memory_space=pltpu.MemorySpace.SMEM),
        ],
        out_specs=[
            pl.BlockSpec((B, D), lambda: (0, 0)),
            pl.BlockSpec((B, C), lambda: (0, 0)),
            pl.BlockSpec((N, D), lambda: (0, 0)),
            pl.BlockSpec((N, 2 * C), lambda: (0, 0)),
            pl.BlockSpec((N, D), lambda: (0, 0)),
        ],
        out_shape=[
            jax.ShapeDtypeStruct((B, D), jnp.float32),
            jax.ShapeDtypeStruct((B, C), jnp.float32),
            jax.ShapeDtypeStruct((N, D), jnp.float32),
            jax.ShapeDtypeStruct((N, 2 * C), jnp.float32),
            jax.ShapeDtypeStruct((N, D), jnp.bfloat16),
        ],
        compiler_params=pltpu.CompilerParams(
            vmem_limit_bytes=100 * 1024 * 1024),
    )(features, predictions, fea_bank, score_bank, trg_idx)


def _topk_body(x_ref, bank_ref, idx_ref, runk_ref, *, bm):
    j = pl.program_id(1)

    s = jax.lax.dot_general(
        x_ref[...], bank_ref[...], (((1,), (1,)), ((), ())),
        preferred_element_type=jnp.float32)
    bits = jax.lax.bitcast_convert_type(s, jnp.int32)
    skey = jnp.where(bits >= 0, bits, bits ^ jnp.int32(0x7FFFFFFF))
    cols = jax.lax.broadcasted_iota(jnp.int32, (bm, BN), 1)
    inv = jnp.int32(N - 1) - (cols + j * BN)
    key = (skey & jnp.int32(-16384)) | inv

    w2 = BN // PAGES
    m1 = key[:, 0:w2]
    m2 = jnp.full((bm, w2), IMIN, jnp.int32)
    for p in range(1, PAGES):
        pk = key[:, p * w2:(p + 1) * w2]
        m2 = jnp.maximum(m2, jnp.minimum(m1, pk))
        m1 = jnp.maximum(m1, pk)

    @pl.when(j == 0)
    def _():
        runk_ref[...] = jnp.full((bm, TOPK), IMIN, jnp.int32)

    cand = jnp.concatenate([runk_ref[...], m1, m2], axis=1)
    nk = []
    for _ in range(TOPK):
        m = jnp.max(cand, axis=1, keepdims=True)
        nk.append(m)
        cand = jnp.where(cand == m, IMIN, cand)
    runk_ref[...] = jnp.concatenate(nk, axis=1)

    @pl.when(j == NT - 1)
    def _():
        idx_ref[...] = jnp.int32(N - 1) - (runk_ref[...] & jnp.int32(16383))


def _topk_stage(x, bank, bm):
    m = x.shape[0]
    return pl.pallas_call(
        functools.partial(_topk_body, bm=bm),
        grid=(m // bm, NT),
        in_specs=[
            pl.BlockSpec((bm, D), lambda i, j: (i, 0)),
            pl.BlockSpec((BN, D), lambda i, j: (j, 0)),
        ],
        out_specs=pl.BlockSpec((bm, TOPK), lambda i, j: (i, 0)),
        out_shape=jax.ShapeDtypeStruct((m, TOPK), jnp.int32),
        scratch_shapes=[
            pltpu.VMEM((bm, TOPK), jnp.int32),
        ],
        compiler_params=pltpu.CompilerParams(
            dimension_semantics=("arbitrary", "arbitrary"),
            vmem_limit_bytes=100 * 1024 * 1024),
    )(x, bank)


def _sc_gather(bank, flat_idx, window):
    num_idx = flat_idx.shape[1]
    d = bank.shape[1]
    mesh = plsc.VectorSubcoreMesh(core_axis_name="core",
                                  subcore_axis_name="subcore")

    @functools.partial(
        pl.kernel,
        out_type=jax.ShapeDtypeStruct((num_idx, d), bank.dtype),
        mesh=mesh)
    def _gather_kernel(bank_hbm, idx_hbm, out_hbm):
        def body(i_vmem, o_vmem):
            pltpu.sync_copy(bank_hbm.at[i_vmem.at[0]], o_vmem)

        pltpu.emit_pipeline(
            body,
            grid=(num_idx // window,),
            in_specs=[pl.BlockSpec((1, window), lambda i: (0, i))],
            out_specs=[pl.BlockSpec((window, d), lambda i: (i, 0))],
            core_axis_name=("core", "subcore"),
            dimension_semantics=(pltpu.PARALLEL,),
        )(idx_hbm, out_hbm)

    return _gather_kernel(bank, flat_idx)


def _loss_body(sm_ref, snear_ref, snn_ref, inp5_ref, inp25_ref,
               idxnn_ref, trg5_ref, out_ref):
    snn = snn_ref[...][:, 0:C]
    t_logt_nn = jnp.where(snn > 0,
                          snn * jnp.log(jnp.where(snn > 0, snn, 1.0)), 0.0)
    kl1 = jnp.sum(t_logt_nn - snn * inp25_ref[...], axis=1, keepdims=True)
    term1 = jnp.sum(kl1) * (0.1 / B)

    sn = snear_ref[...][:, 0:C]
    t_logt_n = jnp.where(sn > 0,
                         sn * jnp.log(jnp.where(sn > 0, sn, 1.0)), 0.0)
    kl2 = jnp.sum(t_logt_n - sn * inp5_ref[...], axis=1, keepdims=True)

    nn = idxnn_ref[...][:, 1:]
    match = jnp.sum((nn == trg5_ref[...]).astype(jnp.float32),
                    axis=1, keepdims=True)
    weight = jnp.where(match > 0.0, match, 0.1)
    term2 = jnp.sum(kl2 * weight) / B

    sm = sm_ref[...]
    msm = jnp.mean(sm, axis=0, keepdims=True)
    gentropy = jnp.sum(msm * jnp.log(msm + EPS))

    out_ref[...] = jnp.broadcast_to(term1 + term2 + gentropy, (1, 1))


def _loss(sm, s_near, s_nn, inp5, inp25, idx_nn6, trg5):
    return pl.pallas_call(
        _loss_body,
        grid=(),
        in_specs=[
            pl.BlockSpec((B, C), lambda: (0, 0)),
            pl.BlockSpec((B * K, 2 * C), lambda: (0, 0)),
            pl.BlockSpec((B * K * K, 2 * C), lambda: (0, 0)),
            pl.BlockSpec((B * K, C), lambda: (0, 0)),
            pl.BlockSpec((B * K * K, C), lambda: (0, 0)),
            pl.BlockSpec((B * K, TOPK), lambda: (0, 0)),
            pl.BlockSpec((B * K, 1), lambda: (0, 0)),
        ],
        out_specs=pl.BlockSpec((1, 1), lambda: (0, 0)),
        out_shape=jax.ShapeDtypeStruct((1, 1), jnp.float32),
        compiler_params=pltpu.CompilerParams(
            vmem_limit_bytes=100 * 1024 * 1024),
    )(sm, s_near, s_nn, inp5, inp25, idx_nn6, trg5)


def kernel(features, predictions, fea_bank, score_bank, trg_idx):
    q, sm, fea_new, score_new, fea_bf = _prep(
        features, predictions, fea_bank, score_bank, trg_idx)

    idx_near6 = _topk_stage(q.astype(jnp.bfloat16), fea_bf, bm=256)
    idx_near = idx_near6[:, 1:]
    flat_near = idx_near.reshape(1, B * K)

    fea_near = _sc_gather(fea_new, flat_near, window=128)
    s_near = _sc_gather(score_new, flat_near, window=128)

    idx_nn6 = _topk_stage(fea_near.astype(jnp.bfloat16), fea_bf, bm=256)
    idx_nn = idx_nn6[:, 1:]
    s_nn = _sc_gather(score_new, idx_nn.reshape(1, B * K * K), window=256)

    inp5 = jnp.broadcast_to(sm[:, None, :], (B, K, C)).reshape(B * K, C)
    inp25 = jnp.broadcast_to(sm[:, None, :], (B, K * K, C)).reshape(B * K * K, C)
    trg5 = jnp.broadcast_to(trg_idx[:, None, None], (B, K, 1)).reshape(B * K, 1)

    loss = _loss(sm, s_near, s_nn, inp5, inp25, idx_nn6, trg5)
    return loss.reshape(())


# speedup vs baseline: 1.0671x; 1.0671x over previous
import functools

import jax
import jax.numpy as jnp
from jax.experimental import pallas as pl
from jax.experimental.pallas import tpu as pltpu
from jax.experimental.pallas import tpu_sc as plsc

B, N, D, C = 512, 16384, 256, 64
K = 5
EPS = 1e-05

BN = 4096
NT = N // BN
TOPK = K + 1
PAGES = 32
IMIN = -2**31


def _prep_body(feat_ref, pred_ref, fbank_ref, sbank_ref, trg_ref,
               qbf_ref, sm_ref, fbf_ref, snew_ref, fnew_ref):
    f = feat_ref[...]
    nrm = jnp.maximum(jnp.sqrt(jnp.sum(f * f, axis=1, keepdims=True)), 1e-12)
    q = f / nrm
    qbf_ref[...] = q.astype(jnp.bfloat16)
    p = pred_ref[...]
    p = p - jnp.max(p, axis=1, keepdims=True)
    e = jnp.exp(p)
    sm = e / jnp.sum(e, axis=1, keepdims=True)
    sm_ref[...] = sm
    fnew_ref[0:N, :] = fbank_ref[...]
    fnew_ref[N:N + B, :] = q
    snew_ref[:, 0:C] = sbank_ref[...]
    snew_ref[:, C:2 * C] = jnp.zeros((N, C), jnp.float32)

    def body(b, carry):
        idx = trg_ref[b]
        fnew_ref[pl.ds(idx, 1), :] = fnew_ref[pl.ds(N + b, 1), :]
        snew_ref[pl.ds(idx, 1), 0:C] = sm_ref[pl.ds(b, 1), :]
        return carry

    jax.lax.fori_loop(0, B, body, 0)
    fbf_ref[...] = fnew_ref[0:N, :].astype(jnp.bfloat16)


def _prep(features, predictions, fea_bank, score_bank, trg_idx):
    return pl.pallas_call(
        _prep_body,
        grid=(),
        in_specs=[
            pl.BlockSpec((B, D), lambda: (0, 0)),
            pl.BlockSpec((B, C), lambda: (0, 0)),
            pl.BlockSpec((N, D), lambda: (0, 0)),
            pl.BlockSpec((N, C), lambda: (0, 0)),
            pl.BlockSpec(memory_space=pltpu.MemorySpace.SMEM),
        ],
        out_specs=[
            pl.BlockSpec((B, D), lambda: (0, 0)),
            pl.BlockSpec((B, C), lambda: (0, 0)),
            pl.BlockSpec((N, D), lambda: (0, 0)),
            pl.BlockSpec((N, 2 * C), lambda: (0, 0)),
            pl.BlockSpec((N + B, D), lambda: (0, 0)),
        ],
        out_shape=[
            jax.ShapeDtypeStruct((B, D), jnp.bfloat16),
            jax.ShapeDtypeStruct((B, C), jnp.float32),
            jax.ShapeDtypeStruct((N, D), jnp.bfloat16),
            jax.ShapeDtypeStruct((N, 2 * C), jnp.float32),
            jax.ShapeDtypeStruct((N + B, D), jnp.float32),
        ],
        compiler_params=pltpu.CompilerParams(
            vmem_limit_bytes=110 * 1024 * 1024),
    )(features, predictions, fea_bank, score_bank, trg_idx)


def _topk_update(buf_ref, runk_ref, tile, valid, reset, bm):
    s = buf_ref[...]
    bits = jax.lax.bitcast_convert_type(s, jnp.int32)
    w2 = BN // PAGES
    lane = jax.lax.broadcasted_iota(jnp.int32, (bm, w2), 1)
    base = jnp.int32(N - 1) - tile * BN

    def page_key(p):
        pb = bits[:, p * w2:(p + 1) * w2]
        sk = jnp.where(pb >= 0, pb, pb ^ jnp.int32(0x7FFFFFFF))
        return (sk & jnp.int32(-16384)) | ((base - p * w2) - lane)

    m1 = page_key(0)
    m2 = jnp.full((bm, w2), IMIN, jnp.int32)
    for p in range(1, PAGES):
        pk = page_key(p)
        m2 = jnp.maximum(m2, jnp.minimum(m1, pk))
        m1 = jnp.maximum(m1, pk)
    m1 = jnp.where(valid, m1, IMIN)
    m2 = jnp.where(valid, m2, IMIN)
    prev = jnp.where(reset, jnp.full((bm, TOPK), IMIN, jnp.int32),
                     runk_ref[...])
    cand = jnp.concatenate([prev, m1, m2], axis=1)
    nk = []
    for _ in range(TOPK):
        m = jnp.max(cand, axis=1, keepdims=True)
        nk.append(m)
        cand = jnp.where(cand == m, IMIN, cand)
    top = jnp.concatenate(nk, axis=1)
    runk_ref[...] = top
    return jnp.int32(N - 1) - (top & jnp.int32(16383))


def _topk_body(x_ref, banka_ref, bankb_ref, idx_ref, bufa_ref, bufb_ref,
               runk_ref, *, bm, nrow):
    u = pl.program_id(0)
    wa = 2 * u
    ta = jax.lax.rem(wa, NT)
    tb = jax.lax.rem(wa + 1, NT)
    tprev = jax.lax.rem(wa - 1, NT)

    bufa_ref[...] = jax.lax.dot_general(
        x_ref[...], banka_ref[...], (((1,), (1,)), ((), ())),
        preferred_element_type=jnp.float32)

    idx_ref[...] = _topk_update(
        bufb_ref, runk_ref, tprev, u > 0, jnp.bool_(False), bm)

    bufb_ref[...] = jax.lax.dot_general(
        x_ref[...], bankb_ref[...], (((1,), (1,)), ((), ())),
        preferred_element_type=jnp.float32)

    _topk_update(
        bufa_ref, runk_ref, ta, wa < nrow * NT, ta == 0, bm)


def _topk_stage(x, bank, bm):
    m = x.shape[0]
    nrow = m // bm
    steps = (nrow * NT) // 2 + 1
    return pl.pallas_call(
        functools.partial(_topk_body, bm=bm, nrow=nrow),
        grid=(steps,),
        in_specs=[
            pl.BlockSpec(
                (bm, D), lambda u: (jnp.minimum(u // (NT // 2), nrow - 1), 0)),
            pl.BlockSpec((BN, D), lambda u: (jax.lax.rem(2 * u, NT), 0)),
            pl.BlockSpec((BN, D), lambda u: (jax.lax.rem(2 * u + 1, NT), 0)),
        ],
        out_specs=pl.BlockSpec(
            (bm, TOPK), lambda u: (jnp.maximum(2 * u - 1, 0) // NT, 0)),
        out_shape=jax.ShapeDtypeStruct((m, TOPK), jnp.int32),
        scratch_shapes=[
            pltpu.VMEM((bm, BN), jnp.float32),
            pltpu.VMEM((bm, BN), jnp.float32),
            pltpu.VMEM((bm, TOPK), jnp.int32),
        ],
        compiler_params=pltpu.CompilerParams(
            dimension_semantics=("arbitrary",),
            vmem_limit_bytes=110 * 1024 * 1024),
    )(x, bank, bank)


def _sc_gather(bank, flat_idx, window):
    num_idx = flat_idx.shape[1]
    d = bank.shape[1]
    mesh = plsc.VectorSubcoreMesh(core_axis_name="core",
                                  subcore_axis_name="subcore")

    @functools.partial(
        pl.kernel,
        out_type=jax.ShapeDtypeStruct((num_idx, d), bank.dtype),
        mesh=mesh)
    def _gather_kernel(bank_hbm, idx_hbm, out_hbm):
        def body(i_vmem, o_vmem):
            pltpu.sync_copy(bank_hbm.at[i_vmem.at[0]], o_vmem)

        pltpu.emit_pipeline(
            body,
            grid=(num_idx // window,),
            in_specs=[pl.BlockSpec((1, window), lambda i: (0, i))],
            out_specs=[pl.BlockSpec((window, d), lambda i: (i, 0))],
            core_axis_name=("core", "subcore"),
            dimension_semantics=(pltpu.PARALLEL,),
        )(idx_hbm, out_hbm)

    return _gather_kernel(bank, flat_idx)


def _loss_body(sm_ref, snear_ref, snn_ref, inp5_ref, inp25_ref,
               idxnn_ref, trg5_ref, out_ref):
    snn = snn_ref[...][:, 0:C]
    t_logt_nn = jnp.where(snn > 0,
                          snn * jnp.log(jnp.where(snn > 0, snn, 1.0)), 0.0)
    kl1 = jnp.sum(t_logt_nn - snn * inp25_ref[...], axis=1, keepdims=True)
    term1 = jnp.sum(kl1) * (0.1 / B)

    sn = snear_ref[...][:, 0:C]
    t_logt_n = jnp.where(sn > 0,
                         sn * jnp.log(jnp.where(sn > 0, sn, 1.0)), 0.0)
    kl2 = jnp.sum(t_logt_n - sn * inp5_ref[...], axis=1, keepdims=True)

    nn = idxnn_ref[...][:, 1:]
    match = jnp.sum((nn == trg5_ref[...]).astype(jnp.float32),
                    axis=1, keepdims=True)
    weight = jnp.where(match > 0.0, match, 0.1)
    term2 = jnp.sum(kl2 * weight) / B

    sm = sm_ref[...]
    msm = jnp.mean(sm, axis=0, keepdims=True)
    gentropy = jnp.sum(msm * jnp.log(msm + EPS))

    out_ref[...] = jnp.broadcast_to(term1 + term2 + gentropy, (1, 1))


def _loss(sm, s_near, s_nn, inp5, inp25, idx_nn6, trg5):
    return pl.pallas_call(
        _loss_body,
        grid=(),
        in_specs=[
            pl.BlockSpec((B, C), lambda: (0, 0)),
            pl.BlockSpec((B * K, 2 * C), lambda: (0, 0)),
            pl.BlockSpec((B * K * K, 2 * C), lambda: (0, 0)),
            pl.BlockSpec((B * K, C), lambda: (0, 0)),
            pl.BlockSpec((B * K * K, C), lambda: (0, 0)),
            pl.BlockSpec((B * K, TOPK), lambda: (0, 0)),
            pl.BlockSpec((B * K, 1), lambda: (0, 0)),
        ],
        out_specs=pl.BlockSpec((1, 1), lambda: (0, 0)),
        out_shape=jax.ShapeDtypeStruct((1, 1), jnp.float32),
        compiler_params=pltpu.CompilerParams(
            vmem_limit_bytes=110 * 1024 * 1024),
    )(sm, s_near, s_nn, inp5, inp25, idx_nn6, trg5)


def kernel(features, predictions, fea_bank, score_bank, trg_idx):
    q_bf, sm, fea_bf, score_new, fea_new = _prep(
        features, predictions, fea_bank, score_bank, trg_idx)

    idx_near6 = _topk_stage(q_bf, fea_bf, bm=256)
    idx_near = idx_near6[:, 1:]
    flat_near = idx_near.reshape(1, B * K)

    fea_near = _sc_gather(
        fea_new, flat_near, window=128).astype(jnp.bfloat16)
    s_near = _sc_gather(score_new, flat_near, window=128)

    idx_nn6 = _topk_stage(fea_near, fea_bf, bm=256)
    idx_nn = idx_nn6[:, 1:]
    s_nn = _sc_gather(score_new, idx_nn.reshape(1, B * K * K), window=256)

    inp5 = jnp.broadcast_to(sm[:, None, :], (B, K, C)).reshape(B * K, C)
    inp25 = jnp.broadcast_to(sm[:, None, :], (B, K * K, C)).reshape(B * K * K, C)
    trg5 = jnp.broadcast_to(trg_idx[:, None, None], (B, K, 1)).reshape(B * K, 1)

    loss = _loss(sm, s_near, s_nn, inp5, inp25, idx_nn6, trg5)
    return loss.reshape(())
